# baseline (speedup 1.0000x reference)
import functools

import jax
import jax.numpy as jnp
from jax import lax
from jax.experimental import pallas as pl
from jax.experimental.pallas import tpu as pltpu
from jax.experimental.pallas import tpu_sc as plsc

N = 1048576
B = 1024
LANES = 16
NC = 2
NS = 16
NW = NC * NS
PER_TILE = N // NW
TBL = LANES * B
UNROLL = 4
NG = PER_TILE // 16

C_SCALE = float(1 << 25)
E_SCALE = float(1 << 19)


def _sc_body(risk_hbm, phase_hbm, cens_hbm, ht_hbm, sums_hbm,
             rv_v, pv_v, cv_v, ht_v, st_v, ex_v, sh_sq,
             sem1, sem2, sem3):
    cid = lax.axis_index("c")
    sid = lax.axis_index("s")
    wid = sid * NC + cid
    mirror = sid * NC + (1 - cid)
    base_elem = wid * PER_TILE

    cp1 = pltpu.make_async_copy(
        risk_hbm.at[pl.ds(base_elem, PER_TILE)], rv_v, sem1)
    cpo = pltpu.make_async_copy(
        risk_hbm.at[pl.ds(mirror * PER_TILE, PER_TILE)], cv_v, sem3)
    cp1.start()
    cpo.start()

    @plsc.parallel_loop(0, TBL // 16, unroll=8)
    def _(i):
        l = lax.shift_right_logical(i, 6)
        r = lax.shift_right_logical(i, 3) & 7
        k = i & 7
        ht_v[l, r, pl.ds(k * 16, 16)] = jnp.zeros((16,), jnp.int32)

    cp1.wait()

    @plsc.parallel_loop(0, NG, unroll=8, carry=jnp.zeros((16,), jnp.float32))
    def acc_own(i, acc):
        a = rv_v[pl.ds(i * 16, 16)]
        return acc + a * a

    cpo.wait()

    @plsc.parallel_loop(0, NG, unroll=8, carry=acc_own)
    def acc_sq(i, acc):
        b = cv_v[pl.ds(i * 16, 16)]
        return acc + b * b

    cp2 = pltpu.make_async_copy(
        phase_hbm.at[pl.ds(base_elem, PER_TILE)], pv_v, sem2)
    cp3 = pltpu.make_async_copy(
        cens_hbm.at[pl.ds(base_elem, PER_TILE)], cv_v, sem3)
    cp2.start()
    cp3.start()

    st_v[pl.ds(0, 16)] = acc_sq
    pltpu.sync_copy(st_v.at[pl.ds(0, 16)], sh_sq.at[pl.ds(sid * 16, 16)])
    plsc.subcore_barrier()
    pltpu.sync_copy(sh_sq, ex_v)
    tot = ex_v[pl.ds(0, 16)]
    for t in range(1, NS):
        tot = tot + ex_v[pl.ds(t * 16, 16)]
    sumsq = jnp.sum(tot)
    sq_vec = jnp.full((16,), sumsq, jnp.float32)

    seed = 0x5F3759DF - lax.shift_right_logical(
        plsc.bitcast(sq_vec, jnp.int32), 1)
    y = plsc.bitcast(seed, jnp.float32)
    for _ in range(4):
        y = y * (1.5 - 0.5 * sq_vec * y * y)
    inv = y

    lane = lax.broadcasted_iota(jnp.int32, (16,), 0)
    cp2.wait()
    cp3.wait()

    zz = (jnp.zeros((16,), jnp.float32), jnp.zeros((16,), jnp.float32))

    @plsc.parallel_loop(0, NG, unroll=UNROLL, carry=zz)
    def accs(i, carry):
        scr, nev = carry
        sl = pl.ds(i * 16, 16)
        r = rv_v[sl]
        c = cv_v[sl]
        e = jnp.exp(r * inv)
        b = jnp.minimum((pv_v[sl] * float(B)).astype(jnp.int32), B - 1)
        val = (c * C_SCALE + e * E_SCALE).astype(jnp.int32)
        plsc.addupdate_scatter(
            ht_v, [lane, lax.shift_right_logical(b, 7), b & 127], val)
        return (scr + c * r, nev + c)

    st_v[pl.ds(0, 16)] = accs[0]
    st_v[pl.ds(16, 16)] = accs[1]
    st_v[pl.ds(32, 16)] = sq_vec
    pltpu.sync_copy(ht_v, ht_hbm.at[pl.ds(wid * LANES, LANES)])
    pltpu.sync_copy(st_v, sums_hbm.at[wid])


_sc_hist = functools.partial(
    pl.kernel,
    out_type=[
        jax.ShapeDtypeStruct((NW * LANES, B // 128, 128), jnp.int32),
        jax.ShapeDtypeStruct((NW, 48), jnp.float32),
    ],
    mesh=plsc.VectorSubcoreMesh(core_axis_name="c", subcore_axis_name="s"),
    compiler_params=pltpu.CompilerParams(needs_layout_passes=False),
    scratch_types=[
        pltpu.VMEM((PER_TILE,), jnp.float32),
        pltpu.VMEM((PER_TILE,), jnp.float32),
        pltpu.VMEM((PER_TILE,), jnp.float32),
        pltpu.VMEM((LANES, B // 128, 128), jnp.int32),
        pltpu.VMEM((48,), jnp.float32),
        pltpu.VMEM((NS * 16,), jnp.float32),
        pltpu.VMEM_SHARED((NS * 16,), jnp.float32),
        pltpu.SemaphoreType.DMA,
        pltpu.SemaphoreType.DMA,
        pltpu.SemaphoreType.DMA,
    ],
)(_sc_body)


def _stage3_body(ht_ref, sums_ref, out_ref):
    v = ht_ref[...]
    m_part = lax.shift_right_logical(v, 25)
    e_part = v - lax.shift_left(m_part, 25)
    E = jnp.sum(e_part.astype(jnp.float32), axis=0) * (1.0 / E_SCALE)
    M = jnp.sum(m_part.astype(jnp.float32), axis=0)

    s = sums_ref[...]
    col = lax.broadcasted_iota(jnp.int32, (32, 48), 1)
    scr = jnp.sum(jnp.where(col < 16, s, 0.0))
    nev = jnp.sum(jnp.where((col >= 16) & (col < 32), s, 0.0))
    sumsq = jnp.sum(jnp.where(col >= 32, s, 0.0)) * (1.0 / 512.0)
    invn = 1.0 / jnp.maximum(jnp.sqrt(sumsq), 1e-12)

    cp = lax.broadcasted_iota(jnp.int32, (128, 128), 0)
    cc = lax.broadcasted_iota(jnp.int32, (128, 128), 1)
    upper = jnp.where(cp >= cc, 1.0, 0.0)
    S = lax.dot_general(E, upper, (((1,), (0,)), ((), ())),
                        preferred_element_type=jnp.float32,
                        precision=lax.Precision.HIGHEST)
    T = jnp.sum(E, axis=1, keepdims=True)
    rr = lax.broadcasted_iota(jnp.int32, (8, 8), 0)
    rp = lax.broadcasted_iota(jnp.int32, (8, 8), 1)
    strict = jnp.where(rp > rr, 1.0, 0.0)
    G = lax.dot_general(strict, T, (((1,), (0,)), ((), ())),
                        preferred_element_type=jnp.float32,
                        precision=lax.Precision.HIGHEST)
    suf = S + G
    base = suf - E
    mid = base + 0.5 * E

    rho = jnp.clip(E / jnp.maximum(2.0 * mid, 1e-30), 0.0, 1.0 - 1e-6)
    r2 = rho * rho
    small = -(r2 / 6 + r2 * r2 / 20 + r2 * r2 * r2 / 42 + r2 * r2 * r2 * r2 / 72)
    big = ((1 + rho) * jnp.log(1 + rho) - (1 - rho) * jnp.log(1 - rho)) \
        / jnp.maximum(2.0 * rho, 1e-30) - 1.0
    corr = jnp.where(rho < 0.5, small, big)
    avg_log = jnp.log(jnp.maximum(mid, 1e-30)) + corr

    S2 = jnp.sum(M * avg_log)
    S1 = scr * invn
    out_ref[0, 0] = -(S1 - S2) / nev


def _stage3(ht3, sums):
    return pl.pallas_call(
        _stage3_body,
        in_specs=[
            pl.BlockSpec(memory_space=pltpu.VMEM),
            pl.BlockSpec(memory_space=pltpu.VMEM),
        ],
        out_specs=pl.BlockSpec(memory_space=pltpu.SMEM),
        out_shape=jax.ShapeDtypeStruct((1, 1), jnp.float32),
    )(ht3, sums)


@jax.jit
def kernel(risk, phase, censors):
    risk1 = risk.reshape(N)
    ht3, sums = _sc_hist(risk1, phase, censors)
    loss = _stage3(ht3, sums)
    return loss.reshape(())


# speedup vs baseline: 1.0585x; 1.0585x over previous
import functools

import jax
import jax.numpy as jnp
from jax import lax
from jax.experimental import pallas as pl
from jax.experimental.pallas import tpu as pltpu
from jax.experimental.pallas import tpu_sc as plsc

N = 1048576
B = 1024
LANES = 16
NC = 2
NS = 16
NW = NC * NS
PER_TILE = N // NW
TBL = LANES * B
UNROLL = 4
NG = PER_TILE // 16

C_SCALE = float(1 << 25)
E_SCALE = float(1 << 19)


def _sc_body(risk_hbm, phase_hbm, cens_hbm, ht_hbm, sums_hbm,
             rv_v, pv_v, cv_v, ht_v, st_v, ex_v, sh_sq,
             sem1, sem2, sem3):
    cid = lax.axis_index("c")
    sid = lax.axis_index("s")
    wid = sid * NC + cid
    mirror = sid * NC + (1 - cid)
    base_elem = wid * PER_TILE

    cp1 = pltpu.make_async_copy(
        risk_hbm.at[pl.ds(base_elem, PER_TILE)], rv_v, sem1)
    cpo = pltpu.make_async_copy(
        risk_hbm.at[pl.ds(mirror * PER_TILE, PER_TILE)], cv_v, sem3)
    cp1.start()
    cpo.start()

    @plsc.parallel_loop(0, TBL // 16, unroll=8)
    def _(i):
        l = lax.shift_right_logical(i, 6)
        r = lax.shift_right_logical(i, 3) & 7
        k = i & 7
        ht_v[l, r, pl.ds(k * 16, 16)] = jnp.zeros((16,), jnp.int32)

    cp1.wait()
    cpo.wait()

    @plsc.parallel_loop(0, NG, unroll=8, carry=jnp.zeros((16,), jnp.float32))
    def acc_sq(i, acc):
        a = rv_v[pl.ds(i * 16, 16)]
        b = cv_v[pl.ds(i * 16, 16)]
        return acc + (a * a + b * b)

    cp2 = pltpu.make_async_copy(
        phase_hbm.at[pl.ds(base_elem, PER_TILE)], pv_v, sem2)
    cp3 = pltpu.make_async_copy(
        cens_hbm.at[pl.ds(base_elem, PER_TILE)], cv_v, sem3)
    cp2.start()
    cp3.start()

    st_v[pl.ds(0, 16)] = acc_sq
    pltpu.sync_copy(st_v.at[pl.ds(0, 16)], sh_sq.at[pl.ds(sid * 16, 16)])
    plsc.subcore_barrier()
    pltpu.sync_copy(sh_sq, ex_v)
    tot = ex_v[pl.ds(0, 16)]
    for t in range(1, NS):
        tot = tot + ex_v[pl.ds(t * 16, 16)]
    sumsq = jnp.sum(tot)
    sq_vec = jnp.full((16,), sumsq, jnp.float32)

    seed = 0x5F3759DF - lax.shift_right_logical(
        plsc.bitcast(sq_vec, jnp.int32), 1)
    y = plsc.bitcast(seed, jnp.float32)
    for _ in range(4):
        y = y * (1.5 - 0.5 * sq_vec * y * y)
    inv = y

    lane = lax.broadcasted_iota(jnp.int32, (16,), 0)
    cp2.wait()
    cp3.wait()

    zz = (jnp.zeros((16,), jnp.float32), jnp.zeros((16,), jnp.float32))

    @plsc.parallel_loop(0, NG, unroll=UNROLL, carry=zz)
    def accs(i, carry):
        scr, nev = carry
        sl = pl.ds(i * 16, 16)
        r = rv_v[sl]
        c = cv_v[sl]
        e = jnp.exp(r * inv)
        b = jnp.minimum((pv_v[sl] * float(B)).astype(jnp.int32), B - 1)
        val = (c * C_SCALE + e * E_SCALE).astype(jnp.int32)
        plsc.addupdate_scatter(
            ht_v, [lane, lax.shift_right_logical(b, 7), b & 127], val)
        return (scr + c * r, nev + c)

    st_v[pl.ds(0, 16)] = accs[0]
    st_v[pl.ds(16, 16)] = accs[1]
    st_v[pl.ds(32, 16)] = sq_vec
    pltpu.sync_copy(ht_v, ht_hbm.at[pl.ds(wid * LANES, LANES)])
    pltpu.sync_copy(st_v, sums_hbm.at[wid])


_sc_hist = functools.partial(
    pl.kernel,
    out_type=[
        jax.ShapeDtypeStruct((NW * LANES, B // 128, 128), jnp.int32),
        jax.ShapeDtypeStruct((NW, 48), jnp.float32),
    ],
    mesh=plsc.VectorSubcoreMesh(core_axis_name="c", subcore_axis_name="s"),
    compiler_params=pltpu.CompilerParams(needs_layout_passes=False),
    scratch_types=[
        pltpu.VMEM((PER_TILE,), jnp.float32),
        pltpu.VMEM((PER_TILE,), jnp.float32),
        pltpu.VMEM((PER_TILE,), jnp.float32),
        pltpu.VMEM((LANES, B // 128, 128), jnp.int32),
        pltpu.VMEM((48,), jnp.float32),
        pltpu.VMEM((NS * 16,), jnp.float32),
        pltpu.VMEM_SHARED((NS * 16,), jnp.float32),
        pltpu.SemaphoreType.DMA,
        pltpu.SemaphoreType.DMA,
        pltpu.SemaphoreType.DMA,
    ],
)(_sc_body)


def _stage3_body(ht_ref, sums_ref, out_ref):
    v = ht_ref[...]
    m_part = lax.shift_right_logical(v, 25)
    e_part = v - lax.shift_left(m_part, 25)
    E = jnp.sum(e_part.astype(jnp.float32), axis=0) * (1.0 / E_SCALE)
    M = jnp.sum(m_part.astype(jnp.float32), axis=0)

    s = sums_ref[...]
    col = lax.broadcasted_iota(jnp.int32, (32, 48), 1)
    scr = jnp.sum(jnp.where(col < 16, s, 0.0))
    nev = jnp.sum(jnp.where((col >= 16) & (col < 32), s, 0.0))
    sumsq = jnp.sum(jnp.where(col >= 32, s, 0.0)) * (1.0 / 512.0)
    invn = 1.0 / jnp.maximum(jnp.sqrt(sumsq), 1e-12)

    cp = lax.broadcasted_iota(jnp.int32, (128, 128), 0)
    cc = lax.broadcasted_iota(jnp.int32, (128, 128), 1)
    upper = jnp.where(cp >= cc, 1.0, 0.0)
    S = lax.dot_general(E, upper, (((1,), (0,)), ((), ())),
                        preferred_element_type=jnp.float32,
                        precision=lax.Precision.HIGHEST)
    T = jnp.sum(E, axis=1, keepdims=True)
    rr = lax.broadcasted_iota(jnp.int32, (8, 8), 0)
    rp = lax.broadcasted_iota(jnp.int32, (8, 8), 1)
    strict = jnp.where(rp > rr, 1.0, 0.0)
    G = lax.dot_general(strict, T, (((1,), (0,)), ((), ())),
                        preferred_element_type=jnp.float32,
                        precision=lax.Precision.HIGHEST)
    suf = S + G
    base = suf - E
    mid = base + 0.5 * E

    rho = jnp.clip(E / jnp.maximum(2.0 * mid, 1e-30), 0.0, 1.0 - 1e-6)
    r2 = rho * rho
    small = -(r2 / 6 + r2 * r2 / 20 + r2 * r2 * r2 / 42 + r2 * r2 * r2 * r2 / 72)
    big = ((1 + rho) * jnp.log(1 + rho) - (1 - rho) * jnp.log(1 - rho)) \
        / jnp.maximum(2.0 * rho, 1e-30) - 1.0
    corr = jnp.where(rho < 0.5, small, big)
    avg_log = jnp.log(jnp.maximum(mid, 1e-30)) + corr

    S2 = jnp.sum(M * avg_log)
    S1 = scr * invn
    out_ref[0, 0] = -(S1 - S2) / nev


def _stage3(ht3, sums):
    return pl.pallas_call(
        _stage3_body,
        in_specs=[
            pl.BlockSpec(memory_space=pltpu.VMEM),
            pl.BlockSpec(memory_space=pltpu.VMEM),
        ],
        out_specs=pl.BlockSpec(memory_space=pltpu.SMEM),
        out_shape=jax.ShapeDtypeStruct((1, 1), jnp.float32),
    )(ht3, sums)


@jax.jit
def kernel(risk, phase, censors):
    risk1 = risk.reshape(N)
    ht3, sums = _sc_hist(risk1, phase, censors)
    loss = _stage3(ht3, sums)
    return loss.reshape(())
